# Initial kernel scaffold; baseline (speedup 1.0000x reference)
import jax
import jax.numpy as jnp
from jax.experimental import pallas as pl


def kernel(x, perm):
    raise NotImplementedError("write your pallas kernel here")


# speedup vs baseline: 2.9955x; 2.9955x over previous
import jax
import jax.numpy as jnp
from jax.experimental import pallas as pl
from jax.experimental.pallas import tpu as pltpu


def _copy_body(perm_ref, x_ref, o_ref):
    o_ref[...] = x_ref[...]


def kernel(x, perm):
    B, C, H, W = x.shape
    grid_spec = pltpu.PrefetchScalarGridSpec(
        num_scalar_prefetch=1,
        grid=(C,),
        in_specs=[
            pl.BlockSpec((B, 1, H, W), lambda i, perm_ref: (0, perm_ref[i], 0, 0)),
        ],
        out_specs=pl.BlockSpec((B, 1, H, W), lambda i, perm_ref: (0, i, 0, 0)),
    )
    return pl.pallas_call(
        _copy_body,
        grid_spec=grid_spec,
        out_shape=jax.ShapeDtypeStruct((B, C, H, W), x.dtype),
    )(perm.astype(jnp.int32), x)
